# initial kernel scaffold (unmeasured)
import jax
import jax.numpy as jnp
from jax import lax
from jax.experimental import pallas as pl
from jax.experimental.pallas import tpu as pltpu

N_DEV = 4


def kernel(x, w_mat):
    m_per, k = x.shape
    _, n_per = w_mat.shape

    def body(x_hbm, w_ref, out_ref, comm_ref, copy_sem, send_sems, recv_sems):
        my_pos = lax.axis_index("i")
        left = lax.rem(my_pos + N_DEV - 1, N_DEV)
        right = lax.rem(my_pos + 1, N_DEV)

        cp = pltpu.make_async_copy(x_hbm, comm_ref.at[0], copy_sem)
        cp.start()
        cp.wait()

        barrier_sem = pltpu.get_barrier_semaphore()
        for nbr in (left, right):
            pl.semaphore_signal(
                barrier_sem, inc=1,
                device_id=(nbr,), device_id_type=pl.DeviceIdType.MESH,
            )
        pl.semaphore_wait(barrier_sem, 2)

        def compute(slot, origin):
            y = jnp.dot(
                comm_ref[slot], w_ref[...], preferred_element_type=jnp.float32
            )
            out_ref[pl.ds(origin * m_per, m_per), :] = jnp.maximum(y, 0.0)

        for h in range(N_DEV - 1):
            rdma = pltpu.make_async_remote_copy(
                src_ref=comm_ref.at[h],
                dst_ref=comm_ref.at[h + 1],
                send_sem=send_sems.at[h],
                recv_sem=recv_sems.at[h],
                device_id=(right,),
                device_id_type=pl.DeviceIdType.MESH,
            )
            rdma.start()
            compute(h, lax.rem(my_pos + N_DEV - h, N_DEV))
            rdma.wait()
        compute(N_DEV - 1, lax.rem(my_pos + 1, N_DEV))

    return pl.pallas_call(
        body,
        out_shape=jax.ShapeDtypeStruct((N_DEV * m_per, n_per), jnp.float32),
        in_specs=[
            pl.BlockSpec(memory_space=pltpu.ANY),
            pl.BlockSpec(memory_space=pltpu.VMEM),
        ],
        out_specs=pl.BlockSpec(memory_space=pltpu.VMEM),
        scratch_shapes=[
            pltpu.VMEM((N_DEV, m_per, k), jnp.float32),
            pltpu.SemaphoreType.DMA,
            pltpu.SemaphoreType.DMA((N_DEV - 1,)),
            pltpu.SemaphoreType.DMA((N_DEV - 1,)),
        ],
        compiler_params=pltpu.CompilerParams(collective_id=0),
    )(x, w_mat)


# baseline (device time: 565361 ns/iter reference)
import jax
import jax.numpy as jnp
from jax import lax
from jax.experimental import pallas as pl
from jax.experimental.pallas import tpu as pltpu

N_DEV = 4


def kernel(x, w_mat):
    m_per, k = x.shape
    _, n_per = w_mat.shape

    def body(x_hbm, w_ref, out_ref, comm_ref, copy_sem, send_sems, recv_sems):
        my_pos = lax.axis_index("i")
        left = lax.rem(my_pos + N_DEV - 1, N_DEV)
        right = lax.rem(my_pos + 1, N_DEV)

        cp = pltpu.make_async_copy(x_hbm, comm_ref.at[0], copy_sem)
        cp.start()
        cp.wait()

        barrier_sem = pltpu.get_barrier_semaphore()
        for nbr in (left, right):
            pl.semaphore_signal(
                barrier_sem, inc=1,
                device_id=(nbr,), device_id_type=pl.DeviceIdType.MESH,
            )
        pl.semaphore_wait(barrier_sem, 2)

        def compute(slot, origin):
            y = jnp.dot(
                comm_ref[slot], w_ref[...], preferred_element_type=jnp.float32
            )
            out_ref[pl.ds(origin * m_per, m_per), :] = jnp.maximum(y, 0.0)

        for h in range(N_DEV - 1):
            rdma = pltpu.make_async_remote_copy(
                src_ref=comm_ref.at[h % 2],
                dst_ref=comm_ref.at[(h + 1) % 2],
                send_sem=send_sems.at[h],
                recv_sem=recv_sems.at[h],
                device_id=(right,),
                device_id_type=pl.DeviceIdType.MESH,
            )
            rdma.start()
            compute(h % 2, lax.rem(my_pos + N_DEV - h, N_DEV))
            rdma.wait()
        compute((N_DEV - 1) % 2, lax.rem(my_pos + 1, N_DEV))

    return pl.pallas_call(
        body,
        out_shape=jax.ShapeDtypeStruct((N_DEV * m_per, n_per), jnp.float32),
        in_specs=[
            pl.BlockSpec(memory_space=pl.ANY),
            pl.BlockSpec(memory_space=pltpu.VMEM),
        ],
        out_specs=pl.BlockSpec(memory_space=pltpu.VMEM),
        scratch_shapes=[
            pltpu.VMEM((2, m_per, k), jnp.float32),
            pltpu.SemaphoreType.DMA,
            pltpu.SemaphoreType.DMA((N_DEV - 1,)),
            pltpu.SemaphoreType.DMA((N_DEV - 1,)),
        ],
        compiler_params=pltpu.CompilerParams(collective_id=0),
    )(x, w_mat)


# device time: 295317 ns/iter; 1.9144x vs baseline; 1.9144x over previous
import jax
import jax.numpy as jnp
from jax import lax
from jax.experimental import pallas as pl
from jax.experimental.pallas import tpu as pltpu

N_DEV = 4
N_HOP = N_DEV - 1
RIGHT = 0
LEFT = 1


def kernel(x, w_mat):
    m_per, k = x.shape
    _, n_per = w_mat.shape
    m_half = m_per // 2

    def body(x_hbm, w_ref, out_ref, comm_ref, copy_sems,
             send_r, recv_r, send_l, recv_l):
        my_pos = lax.axis_index("i")
        left = lax.rem(my_pos + N_DEV - 1, N_DEV)
        right = lax.rem(my_pos + 1, N_DEV)

        cp_t = pltpu.make_async_copy(
            x_hbm.at[pl.ds(0, m_half)], comm_ref.at[0, RIGHT], copy_sems.at[0]
        )
        cp_b = pltpu.make_async_copy(
            x_hbm.at[pl.ds(m_half, m_half)], comm_ref.at[0, LEFT],
            copy_sems.at[1]
        )
        cp_t.start()
        cp_b.start()
        cp_t.wait()
        cp_b.wait()

        barrier_sem = pltpu.get_barrier_semaphore()
        for nbr in (left, right):
            pl.semaphore_signal(
                barrier_sem, inc=1,
                device_id=(nbr,), device_id_type=pl.DeviceIdType.MESH,
            )
        pl.semaphore_wait(barrier_sem, 2)

        def compute(slot, direction, origin):
            off = 0 if direction == RIGHT else m_half
            y = jnp.dot(
                comm_ref[slot, direction], w_ref[...],
                preferred_element_type=jnp.float32,
            )
            out_ref[pl.ds(origin * m_per + off, m_half), :] = jnp.maximum(y, 0.0)

        for h in range(N_HOP):
            s, d = h % 2, (h + 1) % 2
            rdma_r = pltpu.make_async_remote_copy(
                src_ref=comm_ref.at[s, RIGHT],
                dst_ref=comm_ref.at[d, RIGHT],
                send_sem=send_r.at[h],
                recv_sem=recv_r.at[h],
                device_id=(right,),
                device_id_type=pl.DeviceIdType.MESH,
            )
            rdma_l = pltpu.make_async_remote_copy(
                src_ref=comm_ref.at[s, LEFT],
                dst_ref=comm_ref.at[d, LEFT],
                send_sem=send_l.at[h],
                recv_sem=recv_l.at[h],
                device_id=(left,),
                device_id_type=pl.DeviceIdType.MESH,
            )
            rdma_r.start()
            rdma_l.start()
            compute(s, RIGHT, lax.rem(my_pos + N_DEV - h, N_DEV))
            compute(s, LEFT, lax.rem(my_pos + h, N_DEV))
            rdma_r.wait()
            rdma_l.wait()
        s = N_HOP % 2
        compute(s, RIGHT, lax.rem(my_pos + N_DEV - N_HOP, N_DEV))
        compute(s, LEFT, lax.rem(my_pos + N_HOP, N_DEV))

    return pl.pallas_call(
        body,
        out_shape=jax.ShapeDtypeStruct((N_DEV * m_per, n_per), jnp.float32),
        in_specs=[
            pl.BlockSpec(memory_space=pl.ANY),
            pl.BlockSpec(memory_space=pltpu.VMEM),
        ],
        out_specs=pl.BlockSpec(memory_space=pltpu.VMEM),
        scratch_shapes=[
            pltpu.VMEM((2, 2, m_per // 2, k), jnp.float32),
            pltpu.SemaphoreType.DMA((2,)),
            pltpu.SemaphoreType.DMA((N_HOP,)),
            pltpu.SemaphoreType.DMA((N_HOP,)),
            pltpu.SemaphoreType.DMA((N_HOP,)),
            pltpu.SemaphoreType.DMA((N_HOP,)),
        ],
        compiler_params=pltpu.CompilerParams(collective_id=0),
    )(x, w_mat)


# device time: 201379 ns/iter; 2.8074x vs baseline; 1.4665x over previous
import jax
import jax.numpy as jnp
from jax import lax
from jax.experimental import pallas as pl
from jax.experimental.pallas import tpu as pltpu

N_DEV = 4
N_HOP = N_DEV - 1
FROM_LEFT, FROM_RIGHT, FROM_DIAG = 0, 1, 2


def kernel(x, w_mat):
    m, k = x.shape
    _, n_per = w_mat.shape
    kh = k // 2

    def body(x_hbm, w_ref, out_ref, xT, xB, lane_r, lane_l,
             blk_l, blk_r, blk_d, copy_sems,
             wsend_r, wrecv_r, wsend_l, wrecv_l, a2a_send, a2a_recv):
        p = lax.axis_index("i")
        left = lax.rem(p + N_DEV - 1, N_DEV)
        right = lax.rem(p + 1, N_DEV)
        diag = lax.rem(p + 2, N_DEV)

        cp_t = pltpu.make_async_copy(
            x_hbm.at[:, pl.ds(0, kh)], xT, copy_sems.at[0]
        )
        cp_b = pltpu.make_async_copy(
            x_hbm.at[:, pl.ds(kh, kh)], xB, copy_sems.at[1]
        )
        cp_t.start()
        cp_b.start()

        barrier_sem = pltpu.get_barrier_semaphore()
        for nbr in (left, right):
            pl.semaphore_signal(
                barrier_sem, inc=1,
                device_id=(nbr,), device_id_type=pl.DeviceIdType.MESH,
            )
        pl.semaphore_wait(barrier_sem, 2)

        def w_hop(h, src_r, src_l, dst_slot):
            r = pltpu.make_async_remote_copy(
                src_ref=src_r, dst_ref=lane_r.at[dst_slot],
                send_sem=wsend_r.at[h], recv_sem=wrecv_r.at[h],
                device_id=(right,), device_id_type=pl.DeviceIdType.MESH,
            )
            l = pltpu.make_async_remote_copy(
                src_ref=src_l, dst_ref=lane_l.at[dst_slot],
                send_sem=wsend_l.at[h], recv_sem=wrecv_l.at[h],
                device_id=(left,), device_id_type=pl.DeviceIdType.MESH,
            )
            r.start()
            l.start()
            return r, l

        r0, l0 = w_hop(0, w_ref.at[pl.ds(0, kh)], w_ref.at[pl.ds(kh, kh)], 0)
        cp_t.wait()
        cp_b.wait()
        out_ref[pl.ds(p * m, m), :] = jnp.maximum(
            jnp.dot(xT[...], w_ref[pl.ds(0, kh), :],
                    preferred_element_type=jnp.float32)
            + jnp.dot(xB[...], w_ref[pl.ds(kh, kh), :],
                      preferred_element_type=jnp.float32),
            0.0,
        )
        r0.wait()
        l0.wait()

        r1, l1 = w_hop(1, lane_r.at[0], lane_l.at[0], 1)
        blk_l[...] = jnp.dot(
            xT[...], lane_r[0], preferred_element_type=jnp.float32
        )
        blk_r[...] = jnp.dot(
            xB[...], lane_l[0], preferred_element_type=jnp.float32
        )
        r1.wait()
        l1.wait()

        r2, l2 = w_hop(2, lane_r.at[1], lane_l.at[1], 0)
        blk_d[...] = jnp.maximum(
            jnp.dot(xT[...], lane_r[1], preferred_element_type=jnp.float32)
            + jnp.dot(xB[...], lane_l[1], preferred_element_type=jnp.float32),
            0.0,
        )
        send_d = pltpu.make_async_remote_copy(
            src_ref=blk_d, dst_ref=out_ref.at[pl.ds(p * m, m)],
            send_sem=a2a_send.at[FROM_DIAG], recv_sem=a2a_recv.at[FROM_DIAG],
            device_id=(diag,), device_id_type=pl.DeviceIdType.MESH,
        )
        send_d.start()
        r2.wait()
        l2.wait()

        blk_r[...] = jnp.maximum(
            blk_r[...]
            + jnp.dot(xT[...], lane_r[0], preferred_element_type=jnp.float32),
            0.0,
        )
        blk_l[...] = jnp.maximum(
            blk_l[...]
            + jnp.dot(xB[...], lane_l[0], preferred_element_type=jnp.float32),
            0.0,
        )
        send_r = pltpu.make_async_remote_copy(
            src_ref=blk_r, dst_ref=out_ref.at[pl.ds(p * m, m)],
            send_sem=a2a_send.at[FROM_LEFT], recv_sem=a2a_recv.at[FROM_LEFT],
            device_id=(right,), device_id_type=pl.DeviceIdType.MESH,
        )
        send_l = pltpu.make_async_remote_copy(
            src_ref=blk_l, dst_ref=out_ref.at[pl.ds(p * m, m)],
            send_sem=a2a_send.at[FROM_RIGHT], recv_sem=a2a_recv.at[FROM_RIGHT],
            device_id=(left,), device_id_type=pl.DeviceIdType.MESH,
        )
        send_r.start()
        send_l.start()

        recv_left = pltpu.make_async_remote_copy(
            src_ref=blk_r, dst_ref=out_ref.at[pl.ds(left * m, m)],
            send_sem=a2a_send.at[FROM_LEFT], recv_sem=a2a_recv.at[FROM_LEFT],
            device_id=(right,), device_id_type=pl.DeviceIdType.MESH,
        )
        recv_right = pltpu.make_async_remote_copy(
            src_ref=blk_l, dst_ref=out_ref.at[pl.ds(right * m, m)],
            send_sem=a2a_send.at[FROM_RIGHT], recv_sem=a2a_recv.at[FROM_RIGHT],
            device_id=(left,), device_id_type=pl.DeviceIdType.MESH,
        )
        recv_diag = pltpu.make_async_remote_copy(
            src_ref=blk_d, dst_ref=out_ref.at[pl.ds(diag * m, m)],
            send_sem=a2a_send.at[FROM_DIAG], recv_sem=a2a_recv.at[FROM_DIAG],
            device_id=(diag,), device_id_type=pl.DeviceIdType.MESH,
        )
        send_d.wait_send()
        send_r.wait_send()
        send_l.wait_send()
        recv_left.wait_recv()
        recv_right.wait_recv()
        recv_diag.wait_recv()

    return pl.pallas_call(
        body,
        out_shape=jax.ShapeDtypeStruct((N_DEV * m, n_per), jnp.float32),
        in_specs=[
            pl.BlockSpec(memory_space=pl.ANY),
            pl.BlockSpec(memory_space=pltpu.VMEM),
        ],
        out_specs=pl.BlockSpec(memory_space=pltpu.VMEM),
        scratch_shapes=[
            pltpu.VMEM((m, kh), jnp.float32),
            pltpu.VMEM((m, kh), jnp.float32),
            pltpu.VMEM((2, kh, n_per), jnp.float32),
            pltpu.VMEM((2, kh, n_per), jnp.float32),
            pltpu.VMEM((m, n_per), jnp.float32),
            pltpu.VMEM((m, n_per), jnp.float32),
            pltpu.VMEM((m, n_per), jnp.float32),
            pltpu.SemaphoreType.DMA((2,)),
            pltpu.SemaphoreType.DMA((N_HOP,)),
            pltpu.SemaphoreType.DMA((N_HOP,)),
            pltpu.SemaphoreType.DMA((N_HOP,)),
            pltpu.SemaphoreType.DMA((N_HOP,)),
            pltpu.SemaphoreType.DMA((3,)),
            pltpu.SemaphoreType.DMA((3,)),
        ],
        compiler_params=pltpu.CompilerParams(
            collective_id=0,
            vmem_limit_bytes=60 * 1024 * 1024,
        ),
    )(x, w_mat)


# device time: 198113 ns/iter; 2.8537x vs baseline; 1.0165x over previous
import jax
import jax.numpy as jnp
from jax import lax
from jax.experimental import pallas as pl
from jax.experimental.pallas import tpu as pltpu

N_DEV = 4
N_HOP = N_DEV - 1
FROM_LEFT, FROM_RIGHT, FROM_DIAG = 0, 1, 2


def kernel(x, w_mat):
    m, k = x.shape
    _, n_per = w_mat.shape
    kq = k // 4

    def body(x_hbm, w_ref, out_ref, xQ, lane_r, lane_l,
             blk_l, blk_r, blk_d, copy_sems,
             wsend_r, wrecv_r, wsend_l, wrecv_l, a2a_send, a2a_recv):
        p = lax.axis_index("i")
        left = lax.rem(p + N_DEV - 1, N_DEV)
        right = lax.rem(p + 1, N_DEV)
        diag = lax.rem(p + 2, N_DEV)

        copies = []
        for qi in range(4):
            cp = pltpu.make_async_copy(
                x_hbm.at[:, pl.ds(qi * kq, kq)], xQ.at[qi], copy_sems.at[qi]
            )
            cp.start()
            copies.append(cp)

        barrier_sem = pltpu.get_barrier_semaphore()
        for nbr in (left, right):
            pl.semaphore_signal(
                barrier_sem, inc=1,
                device_id=(nbr,), device_id_type=pl.DeviceIdType.MESH,
            )
        pl.semaphore_wait(barrier_sem, 2)

        def w_hop(h, pc, src_r, src_l, dst_slot):
            r = pltpu.make_async_remote_copy(
                src_ref=src_r, dst_ref=lane_r.at[dst_slot, pc],
                send_sem=wsend_r.at[h, pc], recv_sem=wrecv_r.at[h, pc],
                device_id=(right,), device_id_type=pl.DeviceIdType.MESH,
            )
            l = pltpu.make_async_remote_copy(
                src_ref=src_l, dst_ref=lane_l.at[dst_slot, pc],
                send_sem=wsend_l.at[h, pc], recv_sem=wrecv_l.at[h, pc],
                device_id=(left,), device_id_type=pl.DeviceIdType.MESH,
            )
            r.start()
            l.start()
            return r, l

        r00, l00 = w_hop(0, 0, w_ref.at[pl.ds(0, kq)],
                         w_ref.at[pl.ds(2 * kq, kq)], 0)
        r01, l01 = w_hop(0, 1, w_ref.at[pl.ds(kq, kq)],
                         w_ref.at[pl.ds(3 * kq, kq)], 0)

        for cp in copies:
            cp.wait()
        own = jnp.dot(xQ[0], w_ref[pl.ds(0, kq), :],
                      preferred_element_type=jnp.float32)
        for qi in range(1, 4):
            own = own + jnp.dot(xQ[qi], w_ref[pl.ds(qi * kq, kq), :],
                                preferred_element_type=jnp.float32)
        out_ref[pl.ds(p * m, m), :] = jnp.maximum(own, 0.0)

        r00.wait()
        l00.wait()
        r10, l10 = w_hop(1, 0, lane_r.at[0, 0], lane_l.at[0, 0], 1)
        blk_l[...] = jnp.dot(xQ[0], lane_r[0, 0],
                             preferred_element_type=jnp.float32)
        blk_r[...] = jnp.dot(xQ[2], lane_l[0, 0],
                             preferred_element_type=jnp.float32)

        r01.wait()
        l01.wait()
        r11, l11 = w_hop(1, 1, lane_r.at[0, 1], lane_l.at[0, 1], 1)
        blk_l[...] = blk_l[...] + jnp.dot(xQ[1], lane_r[0, 1],
                                          preferred_element_type=jnp.float32)
        blk_r[...] = blk_r[...] + jnp.dot(xQ[3], lane_l[0, 1],
                                          preferred_element_type=jnp.float32)

        r10.wait()
        l10.wait()
        r20, l20 = w_hop(2, 0, lane_r.at[1, 0], lane_l.at[1, 0], 0)
        blk_d[...] = (
            jnp.dot(xQ[0], lane_r[1, 0], preferred_element_type=jnp.float32)
            + jnp.dot(xQ[2], lane_l[1, 0], preferred_element_type=jnp.float32)
        )

        r11.wait()
        l11.wait()
        r21, l21 = w_hop(2, 1, lane_r.at[1, 1], lane_l.at[1, 1], 0)
        blk_d[...] = jnp.maximum(
            blk_d[...]
            + jnp.dot(xQ[1], lane_r[1, 1], preferred_element_type=jnp.float32)
            + jnp.dot(xQ[3], lane_l[1, 1], preferred_element_type=jnp.float32),
            0.0,
        )
        send_d = pltpu.make_async_remote_copy(
            src_ref=blk_d, dst_ref=out_ref.at[pl.ds(p * m, m)],
            send_sem=a2a_send.at[FROM_DIAG], recv_sem=a2a_recv.at[FROM_DIAG],
            device_id=(diag,), device_id_type=pl.DeviceIdType.MESH,
        )
        send_d.start()

        r20.wait()
        l20.wait()
        blk_r[...] = blk_r[...] + jnp.dot(xQ[0], lane_r[0, 0],
                                          preferred_element_type=jnp.float32)
        blk_l[...] = blk_l[...] + jnp.dot(xQ[2], lane_l[0, 0],
                                          preferred_element_type=jnp.float32)

        r21.wait()
        l21.wait()
        blk_r[...] = jnp.maximum(
            blk_r[...] + jnp.dot(xQ[1], lane_r[0, 1],
                                 preferred_element_type=jnp.float32),
            0.0,
        )
        blk_l[...] = jnp.maximum(
            blk_l[...] + jnp.dot(xQ[3], lane_l[0, 1],
                                 preferred_element_type=jnp.float32),
            0.0,
        )
        send_r = pltpu.make_async_remote_copy(
            src_ref=blk_r, dst_ref=out_ref.at[pl.ds(p * m, m)],
            send_sem=a2a_send.at[FROM_LEFT], recv_sem=a2a_recv.at[FROM_LEFT],
            device_id=(right,), device_id_type=pl.DeviceIdType.MESH,
        )
        send_l = pltpu.make_async_remote_copy(
            src_ref=blk_l, dst_ref=out_ref.at[pl.ds(p * m, m)],
            send_sem=a2a_send.at[FROM_RIGHT], recv_sem=a2a_recv.at[FROM_RIGHT],
            device_id=(left,), device_id_type=pl.DeviceIdType.MESH,
        )
        send_r.start()
        send_l.start()

        recv_left = pltpu.make_async_remote_copy(
            src_ref=blk_r, dst_ref=out_ref.at[pl.ds(left * m, m)],
            send_sem=a2a_send.at[FROM_LEFT], recv_sem=a2a_recv.at[FROM_LEFT],
            device_id=(right,), device_id_type=pl.DeviceIdType.MESH,
        )
        recv_right = pltpu.make_async_remote_copy(
            src_ref=blk_l, dst_ref=out_ref.at[pl.ds(right * m, m)],
            send_sem=a2a_send.at[FROM_RIGHT], recv_sem=a2a_recv.at[FROM_RIGHT],
            device_id=(left,), device_id_type=pl.DeviceIdType.MESH,
        )
        recv_diag = pltpu.make_async_remote_copy(
            src_ref=blk_d, dst_ref=out_ref.at[pl.ds(diag * m, m)],
            send_sem=a2a_send.at[FROM_DIAG], recv_sem=a2a_recv.at[FROM_DIAG],
            device_id=(diag,), device_id_type=pl.DeviceIdType.MESH,
        )
        send_d.wait_send()
        send_r.wait_send()
        send_l.wait_send()
        recv_left.wait_recv()
        recv_right.wait_recv()
        recv_diag.wait_recv()

    return pl.pallas_call(
        body,
        out_shape=jax.ShapeDtypeStruct((N_DEV * m, n_per), jnp.float32),
        in_specs=[
            pl.BlockSpec(memory_space=pl.ANY),
            pl.BlockSpec(memory_space=pltpu.VMEM),
        ],
        out_specs=pl.BlockSpec(memory_space=pltpu.VMEM),
        scratch_shapes=[
            pltpu.VMEM((4, m, kq), jnp.float32),
            pltpu.VMEM((2, 2, kq, n_per), jnp.float32),
            pltpu.VMEM((2, 2, kq, n_per), jnp.float32),
            pltpu.VMEM((m, n_per), jnp.float32),
            pltpu.VMEM((m, n_per), jnp.float32),
            pltpu.VMEM((m, n_per), jnp.float32),
            pltpu.SemaphoreType.DMA((4,)),
            pltpu.SemaphoreType.DMA((N_HOP, 2)),
            pltpu.SemaphoreType.DMA((N_HOP, 2)),
            pltpu.SemaphoreType.DMA((N_HOP, 2)),
            pltpu.SemaphoreType.DMA((N_HOP, 2)),
            pltpu.SemaphoreType.DMA((3,)),
            pltpu.SemaphoreType.DMA((3,)),
        ],
        compiler_params=pltpu.CompilerParams(
            collective_id=0,
            vmem_limit_bytes=60 * 1024 * 1024,
        ),
    )(x, w_mat)
